# baseline (speedup 1.0000x reference)
import dataclasses
import functools

import jax
import jax.numpy as jnp
from jax import lax
from jax.experimental import pallas as pl
from jax.experimental.pallas import tpu as pltpu
from jax.experimental.pallas import tpu_sc as plsc

_CHUNK = 96
_NTILES = 32
_SUBCORES = 16
_LANES = 16


def _sc_edge_aggregate(table, srcp, dstp, npad, e_real, with_deg):
    v, width = table.shape
    niter = srcp.shape[0] // (_NTILES * _CHUNK)
    ncr = -(-e_real // _CHUNK)
    rows_per_tile = npad // _SUBCORES
    zeros = jnp.zeros((npad, width), jnp.float32)
    nbuf = 3

    mesh = plsc.VectorSubcoreMesh(core_axis_name="c", subcore_axis_name="s")

    nib = 6
    out_type = [jax.ShapeDtypeStruct((2 * npad, width), jnp.float32)]
    scratch = [pltpu.VMEM((_CHUNK,), jnp.int32)] * (2 * nib)
    scratch += [pltpu.VMEM((_CHUNK, width), jnp.float32)] * nbuf
    scratch += [
        pltpu.VMEM_SHARED((npad, width), jnp.float32),
    ]
    scratch += [pltpu.SemaphoreType.DMA] * (nbuf + nib + nbuf)
    if with_deg:
        out_type.append(jax.ShapeDtypeStruct((_NTILES, npad), jnp.float32))
        scratch.append(pltpu.VMEM((npad,), jnp.float32))

    cp = pltpu.CompilerParams()
    if "needs_layout_passes" in pltpu.CompilerParams.__dataclass_fields__:
        cp = dataclasses.replace(cp, needs_layout_passes=False)

    @functools.partial(pl.kernel, out_type=out_type, mesh=mesh,
                       scratch_types=scratch, compiler_params=cp)
    def agg_kernel(table_hbm, src_hbm, dst_hbm, z_hbm, *refs):
        if with_deg:
            out_hbm, deg_hbm = refs[0], refs[1]
            rest = refs[2:-1]
            ldeg = refs[-1]
        else:
            out_hbm = refs[0]
            rest = refs[1:]
        sidx = rest[0:nib]
        didx = rest[nib:2 * nib]
        rows = rest[2 * nib:2 * nib + nbuf]
        shared = rest[2 * nib + nbuf]
        sems = rest[2 * nib + nbuf + 1:]
        semg = sems[0:nbuf]
        semi = sems[nbuf:nbuf + nib]
        semsc = sems[nbuf + nib:]
        cid = lax.axis_index("c")
        sid = lax.axis_index("s")
        wid = sid * 2 + cid
        cbase = wid * niter * _CHUNK

        def idx_copies(j, q):
            base = cbase + j * _CHUNK
            return (pltpu.make_async_copy(src_hbm.at[pl.ds(base, _CHUNK)],
                                          sidx[q], semi[q]),
                    pltpu.make_async_copy(dst_hbm.at[pl.ds(base, _CHUNK)],
                                          didx[q], semi[q]))

        def gather(q, b):
            return pltpu.make_async_copy(table_hbm.at[sidx[q]],
                                         rows[b], semg[b])

        def scatter(q, b):
            return pltpu.make_async_copy(rows[b], shared.at[didx[q]],
                                         semsc[b])

        myreal = jnp.clip(ncr - wid * niter, 0, niter)

        for q in range(nib - 1):
            @pl.when(q < myreal)
            def _():
                for c in idx_copies(q, q):
                    c.start()

        base_r = sid * rows_per_tile
        pltpu.sync_copy(z_hbm.at[pl.ds(base_r, rows_per_tile)],
                        shared.at[pl.ds(base_r, rows_per_tile)])
        if with_deg:
            zv = jnp.zeros((_LANES,), jnp.float32)

            @pl.loop(0, npad // _LANES)
            def _(i):
                ldeg[pl.ds(i * _LANES, _LANES)] = zv

        plsc.subcore_barrier()

        for js in range(2):
            @pl.when(js < myreal)
            def _():
                for c in idx_copies(js, js):
                    c.wait()
                gather(js, js).start()

        ones = jnp.ones((_LANES,), jnp.float32)

        @pl.loop(0, niter // nib)
        def _(jj):
            for q in range(nib):
                j = jj * nib + q
                r = q % nbuf

                @pl.when(j < myreal)
                def _():
                    gather(q, r).wait()
                    scatter(q, r).start(add=True)
                    if with_deg:
                        for k in range(_CHUNK // _LANES):
                            idxv = didx[q][pl.ds(k * _LANES, _LANES)]
                            plsc.addupdate_scatter(ldeg, [idxv], ones)

                    @pl.when(j + 2 < myreal)
                    def _():
                        for c in idx_copies(j + 2, (q + 2) % nib):
                            c.wait()

                        @pl.when(j >= 1)
                        def _():
                            scatter((q + 5) % nib, (q + 2) % nbuf).wait()

                        gather((q + 2) % nib, (q + 2) % nbuf).start()

                        @pl.when(j + 5 < myreal)
                        def _():
                            for c in idx_copies(j + 5, (q + 5) % nib):
                                c.start()

        for s in range(nbuf):
            @pl.when(myreal > s)
            def _():
                scatter(0, s).wait()

        plsc.subcore_barrier()
        pltpu.sync_copy(shared.at[pl.ds(base_r, rows_per_tile)],
                        out_hbm.at[pl.ds(cid * npad + base_r, rows_per_tile)])
        if with_deg:
            pltpu.sync_copy(ldeg, deg_hbm.at[wid])

    return agg_kernel(table, srcp, dstp, zeros)


def _sum_deg(dp, npad):
    ones = jnp.ones((_NTILES, 1), jnp.float32)
    deg = lax.dot_general(dp, ones, (((0,), (0,)), ((), ())),
                          precision=lax.Precision.HIGHEST,
                          preferred_element_type=jnp.float32)
    return jnp.maximum(deg, 1.0)


def _tc_layer1_body(pa_ref, dp_ref, w_ref, b_ref, h_ref):
    npad = pa_ref.shape[0] // 2
    s = pa_ref[:npad, :] + pa_ref[npad:, :]
    deg = _sum_deg(dp_ref[...], npad)
    z = jnp.dot(s / deg, w_ref[...], precision=lax.Precision.HIGHEST,
                preferred_element_type=jnp.float32)
    h_ref[...] = jnp.maximum(z + b_ref[...], 0.0)


def _tc_layer2_body(pb_ref, dp_ref, batch_ref, w_ref, b_ref, wo_ref, bo_ref,
                    out_ref, *, num_graphs):
    npad = pb_ref.shape[0] // 2
    s = pb_ref[:npad, :] + pb_ref[npad:, :]
    deg = _sum_deg(dp_ref[...], npad)
    h = jnp.maximum(
        jnp.dot(s / deg, w_ref[...], precision=lax.Precision.HIGHEST,
                preferred_element_type=jnp.float32) + b_ref[...], 0.0)
    b = batch_ref[...]
    gids = lax.broadcasted_iota(jnp.int32, (1, num_graphs), 1)
    pt = (b == gids).astype(jnp.float32)
    counts = jnp.maximum(jnp.sum(pt, axis=0), 1.0)
    hg = lax.dot_general(pt, h, (((0,), (0,)), ((), ())),
                         precision=lax.Precision.HIGHEST,
                         preferred_element_type=jnp.float32)
    hg = hg / counts[:, None]
    out_ref[...] = jnp.dot(hg, wo_ref[...], precision=lax.Precision.HIGHEST,
                           preferred_element_type=jnp.float32) + bo_ref[...]


def kernel(x, edge_index, batch, W1, b1, W2, b2, Wout, bout):
    n, d = x.shape
    num_graphs = 64
    npad = ((n + _NTILES * 8 - 1) // (_NTILES * 8)) * (_NTILES * 8)

    e = edge_index.shape[1]
    niter = -(-e // (_NTILES * _CHUNK * 6)) * 6
    epad = _NTILES * _CHUNK * niter
    src = jnp.concatenate(
        [edge_index[0], jnp.zeros((epad - e,), jnp.int32)])
    dst = jnp.concatenate(
        [edge_index[1], jnp.full((epad - e,), n + 8, jnp.int32)])

    pa, dp = _sc_edge_aggregate(x, src, dst, npad, e, with_deg=True)
    h1 = pl.pallas_call(
        _tc_layer1_body,
        out_shape=jax.ShapeDtypeStruct((npad, 128), jnp.float32),
    )(pa, dp, W1, b1)

    (pb,) = _sc_edge_aggregate(h1, src, dst, npad, e, with_deg=False)

    batch_p = jnp.concatenate(
        [batch, jnp.full((npad - n,), num_graphs, jnp.int32)]).reshape(npad, 1)
    out = pl.pallas_call(
        functools.partial(_tc_layer2_body, num_graphs=num_graphs),
        out_shape=jax.ShapeDtypeStruct((num_graphs, 128), jnp.float32),
    )(pb, dp, batch_p, W2, b2, Wout, bout)
    return out


# speedup vs baseline: 1.1821x; 1.1821x over previous
import dataclasses
import functools

import jax
import jax.numpy as jnp
from jax import lax
from jax.experimental import pallas as pl
from jax.experimental.pallas import tpu as pltpu
from jax.experimental.pallas import tpu_sc as plsc

_CHUNK = 80
_NTILES = 32
_SUBCORES = 16
_LANES = 16


def _sc_edge_aggregate(table, srcp, dstp, npad, e_real, with_deg):
    v, width = table.shape
    niter = srcp.shape[0] // (_NTILES * _CHUNK)
    ncr = -(-e_real // _CHUNK)
    rows_per_tile = npad // _SUBCORES
    nbuf = 3

    mesh = plsc.VectorSubcoreMesh(core_axis_name="c", subcore_axis_name="s")

    nib = 6
    out_type = [jax.ShapeDtypeStruct((2 * npad, width), jnp.float32)]
    scratch = [pltpu.VMEM((_CHUNK,), jnp.int32)] * (2 * nib)
    scratch += [pltpu.VMEM((_CHUNK, width), jnp.float32)] * nbuf
    scratch += [
        pltpu.VMEM_SHARED((npad, width), jnp.float32),
    ]
    scratch += [pltpu.SemaphoreType.DMA] * (nbuf + nib + nbuf)
    if with_deg:
        out_type.append(jax.ShapeDtypeStruct((_NTILES, npad), jnp.float32))
        scratch.append(pltpu.VMEM((npad,), jnp.float32))

    cp = pltpu.CompilerParams()
    if "needs_layout_passes" in pltpu.CompilerParams.__dataclass_fields__:
        cp = dataclasses.replace(cp, needs_layout_passes=False)

    @functools.partial(pl.kernel, out_type=out_type, mesh=mesh,
                       scratch_types=scratch, compiler_params=cp)
    def agg_kernel(table_hbm, src_hbm, dst_hbm, *refs):
        if with_deg:
            out_hbm, deg_hbm = refs[0], refs[1]
            rest = refs[2:-1]
            ldeg = refs[-1]
        else:
            out_hbm = refs[0]
            rest = refs[1:]
        sidx = rest[0:nib]
        didx = rest[nib:2 * nib]
        rows = rest[2 * nib:2 * nib + nbuf]
        shared = rest[2 * nib + nbuf]
        sems = rest[2 * nib + nbuf + 1:]
        semg = sems[0:nbuf]
        semi = sems[nbuf:nbuf + nib]
        semsc = sems[nbuf + nib:]
        cid = lax.axis_index("c")
        sid = lax.axis_index("s")
        wid = sid * 2 + cid
        cbase = wid * niter * _CHUNK

        def idx_copies(j, q):
            base = cbase + j * _CHUNK
            return (pltpu.make_async_copy(src_hbm.at[pl.ds(base, _CHUNK)],
                                          sidx[q], semi[q]),
                    pltpu.make_async_copy(dst_hbm.at[pl.ds(base, _CHUNK)],
                                          didx[q], semi[q]))

        def gather(q, b):
            return pltpu.make_async_copy(table_hbm.at[sidx[q]],
                                         rows[b], semg[b])

        def scatter(q, b):
            return pltpu.make_async_copy(rows[b], shared.at[didx[q]],
                                         semsc[b])

        myreal = jnp.clip(ncr - wid * niter, 0, niter)

        for q in range(nib - 1):
            @pl.when(q < myreal)
            def _():
                for c in idx_copies(q, q):
                    c.start()

        zv = jnp.zeros((_LANES,), jnp.float32)

        @pl.loop(0, _CHUNK)
        def _(i):
            for k in range(width // _LANES):
                rows[0][i, pl.ds(k * _LANES, _LANES)] = zv

        base_r = sid * rows_per_tile
        for f in range(rows_per_tile // _CHUNK):
            pltpu.sync_copy(rows[0],
                            shared.at[pl.ds(base_r + f * _CHUNK, _CHUNK)])
        rem = rows_per_tile % _CHUNK
        if rem:
            pltpu.sync_copy(
                rows[0].at[pl.ds(0, rem)],
                shared.at[pl.ds(base_r + rows_per_tile - rem, rem)])
        if with_deg:
            @pl.loop(0, npad // _LANES)
            def _(i):
                ldeg[pl.ds(i * _LANES, _LANES)] = zv

        plsc.subcore_barrier()

        for js in range(2):
            @pl.when(js < myreal)
            def _():
                for c in idx_copies(js, js):
                    c.wait()
                gather(js, js).start()

        ones = jnp.ones((_LANES,), jnp.float32)

        @pl.loop(0, -(-niter // nib))
        def _(jj):
            for q in range(nib):
                j = jj * nib + q
                r = q % nbuf

                @pl.when(j < myreal)
                def _():
                    gather(q, r).wait()
                    scatter(q, r).start(add=True)
                    if with_deg:
                        for k in range(_CHUNK // _LANES):
                            idxv = didx[q][pl.ds(k * _LANES, _LANES)]
                            plsc.addupdate_scatter(ldeg, [idxv], ones)

                    @pl.when(j + 2 < myreal)
                    def _():
                        for c in idx_copies(j + 2, (q + 2) % nib):
                            c.wait()

                        @pl.when(j >= 1)
                        def _():
                            scatter((q + 5) % nib, (q + 2) % nbuf).wait()

                        gather((q + 2) % nib, (q + 2) % nbuf).start()

                        @pl.when(j + 5 < myreal)
                        def _():
                            for c in idx_copies(j + 5, (q + 5) % nib):
                                c.start()

        for s in range(nbuf):
            @pl.when(myreal > s)
            def _():
                scatter(0, s).wait()

        plsc.subcore_barrier()
        pltpu.sync_copy(shared.at[pl.ds(base_r, rows_per_tile)],
                        out_hbm.at[pl.ds(cid * npad + base_r, rows_per_tile)])
        if with_deg:
            pltpu.sync_copy(ldeg, deg_hbm.at[wid])

    return agg_kernel(table, srcp, dstp)


def _tc_layer1_body(pa_ref, dp_ref, w_ref, b_ref, h_ref, dinv_ref):
    npad = pa_ref.shape[0] // 2
    s = pa_ref[:npad, :] + pa_ref[npad:, :]
    ones = jnp.ones((_NTILES, 1), jnp.float32)
    deg = lax.dot_general(dp_ref[...], ones, (((0,), (0,)), ((), ())),
                          preferred_element_type=jnp.float32)
    dinv = 1.0 / jnp.maximum(deg, 1.0)
    dinv_ref[...] = dinv
    z = jnp.dot(s * dinv, w_ref[...], preferred_element_type=jnp.float32)
    h_ref[...] = jnp.maximum(z + b_ref[...], 0.0)


def _tc_layer2_body(pb_ref, dinv_ref, batch_ref, w_ref, b_ref, wo_ref, bo_ref,
                    out_ref, *, num_graphs):
    npad = pb_ref.shape[0] // 2
    s = pb_ref[:npad, :] + pb_ref[npad:, :]
    h = jnp.maximum(
        jnp.dot(s * dinv_ref[...], w_ref[...],
                preferred_element_type=jnp.float32) + b_ref[...], 0.0)
    b = batch_ref[...]
    gids = lax.broadcasted_iota(jnp.int32, (1, num_graphs), 1)
    pt = (b == gids).astype(jnp.float32)
    counts = jnp.maximum(jnp.sum(pt, axis=0), 1.0)
    hg = lax.dot_general(pt, h, (((0,), (0,)), ((), ())),
                         preferred_element_type=jnp.float32)
    hg = hg / counts[:, None]
    out_ref[...] = jnp.dot(hg, wo_ref[...],
                           preferred_element_type=jnp.float32) + bo_ref[...]


def kernel(x, edge_index, batch, W1, b1, W2, b2, Wout, bout):
    n, d = x.shape
    num_graphs = 64
    npad = ((n + _NTILES * 8 - 1) // (_NTILES * 8)) * (_NTILES * 8)

    e = edge_index.shape[1]
    niter = -(-e // (_NTILES * _CHUNK))
    epad = _NTILES * _CHUNK * niter
    src = edge_index[0]
    dst = edge_index[1]
    if epad > e:
        src = jnp.concatenate([src, jnp.zeros((epad - e,), jnp.int32)])
        dst = jnp.concatenate([dst, jnp.full((epad - e,), n + 8, jnp.int32)])

    pa, dp = _sc_edge_aggregate(x, src, dst, npad, e, with_deg=True)
    h1, dinv = pl.pallas_call(
        _tc_layer1_body,
        out_shape=[jax.ShapeDtypeStruct((npad, 128), jnp.float32),
                   jax.ShapeDtypeStruct((npad, 1), jnp.float32)],
    )(pa, dp, W1, b1)

    (pb,) = _sc_edge_aggregate(h1, src, dst, npad, e, with_deg=False)

    batch_p = jnp.concatenate(
        [batch, jnp.full((npad - n,), num_graphs, jnp.int32)]).reshape(npad, 1)
    out = pl.pallas_call(
        functools.partial(_tc_layer2_body, num_graphs=num_graphs),
        out_shape=jax.ShapeDtypeStruct((num_graphs, 128), jnp.float32),
    )(pb, dinv, batch_p, W2, b2, Wout, bout)
    return out
